# Initial kernel scaffold; baseline (speedup 1.0000x reference)
import jax
import jax.numpy as jnp
from jax.experimental import pallas as pl


def kernel(v, input_idx, input_sign):
    raise NotImplementedError("write your pallas kernel here")


# speedup vs baseline: 9.1531x; 9.1531x over previous
import jax
import jax.numpy as jnp
from jax import lax
from jax.experimental import pallas as pl
from jax.experimental.pallas import tpu as pltpu
from jax.experimental.pallas import tpu_sc as plsc

NC = 2
NS = 16
NW = NC * NS

BATCH = 64
N_CLAUSE = 100000
N_SAT = 3
N_VARS = 25000

CHUNK = 4000
N_CHUNKS = N_CLAUSE // CHUNK
B_PER_W = BATCH // NW


def _sc_body(v_hbm, idx_hbm, sign_hbm, out_hbm, v_buf, idx_buf, sign_buf, out_buf):
    wid = lax.axis_index("s") * NC + lax.axis_index("c")
    lane3 = lax.iota(jnp.int32, 16) * 3

    for k in range(B_PER_W):
        b = wid * B_PER_W + k
        pltpu.sync_copy(v_hbm.at[pl.ds(b * N_VARS, N_VARS)], v_buf)

        def chunk_body(ci, carry):
            lit0 = b * (N_CLAUSE * N_SAT) + ci * (CHUNK * 3)
            pltpu.sync_copy(idx_hbm.at[pl.ds(lit0, CHUNK * 3)], idx_buf)
            pltpu.sync_copy(sign_hbm.at[pl.ds(lit0, CHUNK * 3)], sign_buf)

            def body(i, carry2):
                cb = i * 16
                g = lane3 + cb * 3
                i0 = plsc.load_gather(idx_buf, [g])
                i1 = plsc.load_gather(idx_buf, [g + 1])
                i2 = plsc.load_gather(idx_buf, [g + 2])
                s0 = plsc.load_gather(sign_buf, [g])
                s1 = plsc.load_gather(sign_buf, [g + 1])
                s2 = plsc.load_gather(sign_buf, [g + 2])
                v0 = plsc.load_gather(v_buf, [i0])
                v1 = plsc.load_gather(v_buf, [i1])
                v2 = plsc.load_gather(v_buf, [i2])
                m = jnp.maximum(jnp.maximum(v0 * s0, v1 * s1), v2 * s2)
                out_buf[pl.ds(cb, 16)] = 0.5 - 0.5 * m
                return carry2

            lax.fori_loop(0, CHUNK // 16, body, 0)
            pltpu.sync_copy(out_buf, out_hbm.at[pl.ds(b * N_CLAUSE + ci * CHUNK, CHUNK)])
            return carry

        lax.fori_loop(0, N_CHUNKS, chunk_body, 0)


@jax.jit
def kernel(v, input_idx, input_sign):
    idx = input_idx.reshape(BATCH * N_CLAUSE * N_SAT).astype(jnp.int32)
    sgn = input_sign.reshape(BATCH * N_CLAUSE * N_SAT)
    vf = v.reshape(BATCH * N_VARS)
    run = pl.kernel(
        _sc_body,
        out_type=jax.ShapeDtypeStruct((BATCH * N_CLAUSE,), jnp.float32),
        mesh=plsc.VectorSubcoreMesh(core_axis_name="c", subcore_axis_name="s"),
        compiler_params=pltpu.CompilerParams(needs_layout_passes=False),
        scratch_types=[
            pltpu.VMEM((N_VARS,), jnp.float32),
            pltpu.VMEM((CHUNK * 3,), jnp.int32),
            pltpu.VMEM((CHUNK * 3,), jnp.float32),
            pltpu.VMEM((CHUNK,), jnp.float32),
        ],
    )
    return run(vf, idx, sgn).reshape(BATCH, N_CLAUSE)


# speedup vs baseline: 601.1103x; 65.6730x over previous
import jax
import jax.numpy as jnp
from jax import lax
from jax.experimental import pallas as pl
from jax.experimental.pallas import tpu as pltpu
from jax.experimental.pallas import tpu_sc as plsc

NC = 2
NS = 16
NW = NC * NS

BATCH = 64
N_CLAUSE = 100000
N_SAT = 3
N_VARS = 25000

CH = 2048
NFULL = N_CLAUSE // CH
TAIL = N_CLAUSE - NFULL * CH
B_PER_W = BATCH // NW


def _sc_body(v_hbm, idx_hbm, sign_hbm, out_hbm,
             v_buf, idx_buf, sign_buf, out_buf):
    wid = lax.axis_index("s") * NC + lax.axis_index("c")

    def chunk(b, k, c0, nc16, cw):
        pltpu.sync_copy(idx_hbm.at[:, pl.ds(b, 1), pl.ds(c0, cw)],
                        idx_buf.at[:, :, pl.ds(0, cw)])
        pltpu.sync_copy(sign_hbm.at[:, pl.ds(b, 1), pl.ds(c0, cw)],
                        sign_buf.at[:, :, pl.ds(0, cw)])

        def body(i, carry):
            cb = i * 16
            i0 = idx_buf[0, 0, pl.ds(cb, 16)]
            i1 = idx_buf[1, 0, pl.ds(cb, 16)]
            i2 = idx_buf[2, 0, pl.ds(cb, 16)]
            s0 = sign_buf[0, 0, pl.ds(cb, 16)]
            s1 = sign_buf[1, 0, pl.ds(cb, 16)]
            s2 = sign_buf[2, 0, pl.ds(cb, 16)]
            v0 = plsc.load_gather(v_buf.at[k], [i0])
            v1 = plsc.load_gather(v_buf.at[k], [i1])
            v2 = plsc.load_gather(v_buf.at[k], [i2])
            mx = jnp.maximum(jnp.maximum(v0 * s0, v1 * s1), v2 * s2)
            out_buf[0, pl.ds(cb, 16)] = (1.0 - mx) * 0.5
            return carry

        lax.fori_loop(0, nc16, body, 0)
        pltpu.sync_copy(out_buf.at[:, pl.ds(0, cw)],
                        out_hbm.at[pl.ds(b, 1), pl.ds(c0, cw)])

    for k in range(B_PER_W):
        b = wid * B_PER_W + k
        pltpu.sync_copy(v_hbm.at[pl.ds(b * N_VARS, N_VARS)], v_buf.at[k])

        def loop_body(ci, carry):
            chunk(b, k, ci * CH, CH // 16, CH)
            return carry

        lax.fori_loop(0, NFULL, loop_body, 0)
        chunk(b, k, NFULL * CH, TAIL // 16, TAIL)


@jax.jit
def kernel(v, input_idx, input_sign):
    idx = jnp.transpose(input_idx.astype(jnp.int32), (2, 0, 1))
    sgn = jnp.transpose(input_sign, (2, 0, 1))
    vf = v.reshape(BATCH * N_VARS)
    run = pl.kernel(
        _sc_body,
        out_type=jax.ShapeDtypeStruct((BATCH, N_CLAUSE), jnp.float32),
        mesh=plsc.VectorSubcoreMesh(core_axis_name="c", subcore_axis_name="s"),
        compiler_params=pltpu.CompilerParams(
            needs_layout_passes=False, use_tc_tiling_on_sc=False),
        scratch_types=[
            pltpu.VMEM((B_PER_W, N_VARS), jnp.float32),
            pltpu.VMEM((N_SAT, 1, CH), jnp.int32),
            pltpu.VMEM((N_SAT, 1, CH), jnp.float32),
            pltpu.VMEM((1, CH), jnp.float32),
        ],
    )
    return run(vf, idx, sgn)


# speedup vs baseline: 750.8039x; 1.2490x over previous
import jax
import jax.numpy as jnp
from jax import lax
from jax.experimental import pallas as pl
from jax.experimental.pallas import tpu as pltpu
from jax.experimental.pallas import tpu_sc as plsc

NC = 2
NS = 16
NW = NC * NS

BATCH = 64
N_CLAUSE = 100000
N_SAT = 3
N_VARS = 25000

CH = 2048
NFULL = N_CLAUSE // CH
TAIL = N_CLAUSE - NFULL * CH
B_PER_W = BATCH // NW


def _sc_body(v_hbm, idx_hbm, sign_hbm, out_hbm,
             v_buf, idx_buf, sign_buf, out_buf, sem):
    wid = lax.axis_index("s") * NC + lax.axis_index("c")

    def start_in(b, c0, slot, cw):
        pltpu.async_copy(idx_hbm.at[:, pl.ds(b, 1), pl.ds(c0, cw)],
                         idx_buf.at[slot, :, :, pl.ds(0, cw)], sem)
        pltpu.async_copy(sign_hbm.at[:, pl.ds(b, 1), pl.ds(c0, cw)],
                         sign_buf.at[slot, :, :, pl.ds(0, cw)], sem)

    def wait_in(b, c0, slot, cw):
        pltpu.make_async_copy(idx_hbm.at[:, pl.ds(b, 1), pl.ds(c0, cw)],
                              idx_buf.at[slot, :, :, pl.ds(0, cw)], sem).wait()
        pltpu.make_async_copy(sign_hbm.at[:, pl.ds(b, 1), pl.ds(c0, cw)],
                              sign_buf.at[slot, :, :, pl.ds(0, cw)], sem).wait()

    def compute(slot, c0, nc16, cw, b):
        def body(i, carry):
            cb = i * 16
            i0 = idx_buf[slot, 0, 0, pl.ds(cb, 16)]
            i1 = idx_buf[slot, 1, 0, pl.ds(cb, 16)]
            i2 = idx_buf[slot, 2, 0, pl.ds(cb, 16)]
            s0 = sign_buf[slot, 0, 0, pl.ds(cb, 16)]
            s1 = sign_buf[slot, 1, 0, pl.ds(cb, 16)]
            s2 = sign_buf[slot, 2, 0, pl.ds(cb, 16)]
            v0 = plsc.load_gather(v_buf.at[0], [i0])
            v1 = plsc.load_gather(v_buf.at[0], [i1])
            v2 = plsc.load_gather(v_buf.at[0], [i2])
            mx = jnp.maximum(jnp.maximum(v0 * s0, v1 * s1), v2 * s2)
            out_buf[0, pl.ds(cb, 16)] = (1.0 - mx) * 0.5
            return carry

        lax.fori_loop(0, nc16, body, 0)
        pltpu.sync_copy(out_buf.at[:, pl.ds(0, cw)],
                        out_hbm.at[pl.ds(b, 1), pl.ds(c0, cw)])

    for k in range(B_PER_W):
        b = wid * B_PER_W + k
        pltpu.sync_copy(v_hbm.at[pl.ds(b * N_VARS, N_VARS)], v_buf.at[0])
        start_in(b, 0, 0, CH)

        def loop_body(ci, carry):
            slot = ci % 2
            wait_in(b, ci * CH, slot, CH)
            pl.when(ci < NFULL - 1)(
                lambda: start_in(b, (ci + 1) * CH, 1 - slot, CH))
            pl.when(ci == NFULL - 1)(
                lambda: start_in(b, NFULL * CH, 1 - slot, TAIL))
            compute(slot, ci * CH, CH // 16, CH, b)
            return carry

        lax.fori_loop(0, NFULL, loop_body, 0)
        tslot = NFULL % 2
        wait_in(b, NFULL * CH, tslot, TAIL)
        compute(tslot, NFULL * CH, TAIL // 16, TAIL, b)


@jax.jit
def kernel(v, input_idx, input_sign):
    idx = jnp.transpose(input_idx.astype(jnp.int32), (2, 0, 1))
    sgn = jnp.transpose(input_sign, (2, 0, 1))
    vf = v.reshape(BATCH * N_VARS)
    run = pl.kernel(
        _sc_body,
        out_type=jax.ShapeDtypeStruct((BATCH, N_CLAUSE), jnp.float32),
        mesh=plsc.VectorSubcoreMesh(core_axis_name="c", subcore_axis_name="s"),
        compiler_params=pltpu.CompilerParams(
            needs_layout_passes=False, use_tc_tiling_on_sc=False),
        scratch_types=[
            pltpu.VMEM((1, N_VARS), jnp.float32),
            pltpu.VMEM((2, N_SAT, 1, CH), jnp.int32),
            pltpu.VMEM((2, N_SAT, 1, CH), jnp.float32),
            pltpu.VMEM((1, CH), jnp.float32),
            pltpu.SemaphoreType.DMA,
        ],
    )
    return run(vf, idx, sgn)
